# baseline (speedup 1.0000x reference)
import numpy as np
import jax
import jax.numpy as jnp
from jax.experimental import pallas as pl
from jax.experimental.pallas import tpu as pltpu

H = 512
W = 512
NC = 21
BPL = [6, 6, 6, 4, 4]
FM = [16, 8, 4, 2, 1]
CAND_K = 200
MAX_DET = 100
CONF_TH = 0.01
NMS_TH = 0.45
CV, SV = 0.1, 0.2
B = 8

NPRI = sum(f * f * b for f, b in zip(FM, BPL))
NPRI_PAD = 2048
NFLAT = NPRI * (NC - 1)
NFLAT_PAD = 40960
KDIM = 256 * 9


def _priors_np():
    strides_x = [32, 64, 128, 213, 320]
    strides_y = [32, 64, 128, 256, 512]
    expand = H / 256.0
    mins = [10 * expand, 62 * expand, 114 * expand, 166 * expand, 218 * expand]
    maxs = [62 * expand, 114 * expand, 166 * expand, 218 * expand, 270 * expand]
    ars = [[2, 3], [2, 3], [2, 3], [2], [2]]
    pri = []
    for k in range(5):
        fy, fx = int(H / strides_y[k]), int(W / strides_x[k])
        for i in range(fy):
            for j in range(fx):
                cx, cy = (j + 0.5) / fx, (i + 0.5) / fy
                w0, h0 = mins[k] / W, mins[k] / H
                pri.append([cx, cy, w0, h0])
                s2 = float(np.sqrt(mins[k] * maxs[k]))
                pri.append([cx, cy, s2 / W, s2 / H])
                for r in ars[k]:
                    rt = float(np.sqrt(r))
                    pri.append([cx, cy, w0 * rt, h0 / rt])
                    pri.append([cx, cy, w0 / rt, h0 * rt])
    return np.clip(np.array(pri, np.float32), 0.0, 1.0)


_PRI = _priors_np()
assert _PRI.shape[0] == NPRI


def _im2col(feat):
    b, c, f, _ = feat.shape
    xp = jnp.pad(feat, ((0, 0), (0, 0), (1, 1), (1, 1)))
    taps = [xp[:, :, di:di + f, dj:dj + f] for di in range(3) for dj in range(3)]
    x = jnp.stack(taps, axis=1)
    x = x.reshape(b, c * 9, f * f)
    return jnp.transpose(x, (1, 0, 2)).reshape(c * 9, b * f * f)


def _conv_kernel(*refs):
    xs = refs[0:5]
    ws = refs[5:10]
    bs = refs[10:15]
    os_ = refs[15:20]
    for x, w, bb, o in zip(xs, ws, bs, os_):
        acc = jax.lax.dot_general(
            w[...], x[...], (((1,), (0,)), ((), ())),
            preferred_element_type=jnp.float32)
        o[...] = acc + bb[...]


def _head_kernel(cls_ref, reg_ref, pri_ref, sc_ref, box_ref):
    c = cls_ref[...]
    m = c[:, 0:1, :]
    for i in range(1, NC):
        m = jnp.maximum(m, c[:, i:i + 1, :])
    e = jnp.exp(c - m)
    s = e[:, 0:1, :]
    for i in range(1, NC):
        s = s + e[:, i:i + 1, :]
    p = e / s
    sc = p[:, 1:, :]
    sc_ref[...] = jnp.where(sc > CONF_TH, sc, 0.0)

    r = reg_ref[...]
    pcx = pri_ref[0:1, :]
    pcy = pri_ref[1:2, :]
    pw = pri_ref[2:3, :]
    ph = pri_ref[3:4, :]
    cvpw = pri_ref[4:5, :]
    cvph = pri_ref[5:6, :]
    cx = r[:, 0, :] * cvpw + pcx
    cy = r[:, 1, :] * cvph + pcy
    wd = jnp.exp(r[:, 2, :] * SV) * pw
    hg = jnp.exp(r[:, 3, :] * SV) * ph
    box_ref[:, 0, :] = cx - wd / 2.0
    box_ref[:, 1, :] = cy - hg / 2.0
    box_ref[:, 2, :] = cx + wd / 2.0
    box_ref[:, 3, :] = cy + hg / 2.0


NCHUNK = NFLAT_PAD // 128


def _topk_kernel(sc_ref, ts_ref, ti_ref, v_ref):
    v_ref[...] = sc_ref[...]
    kio = jax.lax.broadcasted_iota(jnp.int32, (B, CAND_K), 1)
    cio = jax.lax.broadcasted_iota(jnp.int32, (B, NCHUNK), 1)
    lio = jax.lax.broadcasted_iota(jnp.int32, (1, 128), 1)
    bio = jax.lax.broadcasted_iota(jnp.int32, (B, 1), 0)

    def step(j, carry):
        M, ts, ti = carry
        m = jnp.max(M, axis=1, keepdims=True)
        cmin = jnp.min(jnp.where(M == m, cio, NCHUNK),
                       axis=1, keepdims=True)
        fi_v = jnp.zeros((B, 1), jnp.int32)
        nm_v = jnp.zeros((B, 1), jnp.float32)
        for b in range(B):
            cb = cmin[b, 0]
            rm = m[b, 0]
            row = v_ref[b, pl.ds(cb, 1), :]
            lidx = jnp.min(jnp.where(row == rm, lio, 128))
            masked = jnp.where(lio == lidx, -1.0, row)
            v_ref[b, pl.ds(cb, 1), :] = masked
            fi_v = jnp.where(bio == b, cb * 128 + lidx, fi_v)
            nm_v = jnp.where(bio == b, jnp.max(masked), nm_v)
        M = jnp.where(cio == cmin, nm_v, M)
        sel = kio == j
        ts = jnp.where(sel, m, ts)
        ti = jnp.where(sel, fi_v, ti)
        return M, ts, ti

    M0 = jnp.max(v_ref[...], axis=2)
    _, ts, ti = jax.lax.fori_loop(
        0, CAND_K, step,
        (M0, jnp.zeros((B, CAND_K), jnp.float32),
         jnp.zeros((B, CAND_K), jnp.int32)))
    ts_ref[...] = ts
    ti_ref[...] = ti


def _nms_kernel(ts_ref, ti_ref, box_ref, fb_ref, fl_ref, fs_ref,
                oh_ref, sup_ref, cb_ref):
    ti = ti_ref[...]
    ts = ts_ref[...]
    pidx = ti // (NC - 1)
    label_f = (ti % (NC - 1) + 1).astype(jnp.float32)

    lane = jax.lax.broadcasted_iota(jnp.int32, (B, CAND_K, NPRI_PAD), 2)
    oh_ref[...] = (lane == pidx[:, :, None]).astype(jnp.float32)
    oh = oh_ref[...]
    for c in range(4):
        bc = box_ref[:, c, :][:, None, :]
        cb_ref[:, c, :] = jnp.sum(oh * bc, axis=2)

    nb = [cb_ref[:, c, :] + label_f * 2.0 for c in range(4)]
    ltx = jnp.maximum(nb[0][:, :, None], nb[0][:, None, :])
    lty = jnp.maximum(nb[1][:, :, None], nb[1][:, None, :])
    rbx = jnp.minimum(nb[2][:, :, None], nb[2][:, None, :])
    rby = jnp.minimum(nb[3][:, :, None], nb[3][:, None, :])
    w_i = jnp.clip(rbx - ltx, 0.0)
    h_i = jnp.clip(rby - lty, 0.0)
    inter = w_i * h_i
    area = (nb[2] - nb[0]) * (nb[3] - nb[1])
    iou = inter / (area[:, :, None] + area[:, None, :] - inter + 1e-9)
    jgti = (jax.lax.broadcasted_iota(jnp.int32, (B, CAND_K, CAND_K), 2) >
            jax.lax.broadcasted_iota(jnp.int32, (B, CAND_K, CAND_K), 1))
    sup_ref[...] = jnp.where((iou > NMS_TH) & jgti, 1.0, 0.0)

    iota2 = jax.lax.broadcasted_iota(jnp.int32, (B, CAND_K), 1)

    def nms_step(i, keep):
        row = sup_ref[:, pl.ds(i, 1), :].reshape(B, CAND_K)
        ki = jnp.sum(jnp.where(iota2 == i, keep, 0.0), axis=1, keepdims=True)
        return keep * (1.0 - row * ki)

    keep = jax.lax.fori_loop(0, CAND_K, nms_step,
                             jnp.ones((B, CAND_K), jnp.float32))

    fs_all = ts * keep
    dio = jax.lax.broadcasted_iota(jnp.int32, (B, MAX_DET), 1)

    def sel_step(j, carry):
        v, fs, fl, b0, b1, b2, b3 = carry
        m = jnp.max(v, axis=1, keepdims=True)
        idx = jnp.where(v == m, iota2, CAND_K)
        imin = jnp.min(idx, axis=1, keepdims=True)
        oh2 = (iota2 == imin).astype(jnp.float32)
        sel = dio == j
        fs = jnp.where(sel, m, fs)
        lab = jnp.sum(oh2 * label_f, axis=1, keepdims=True)
        fl = jnp.where(sel, lab, fl)
        outs = []
        for c, acc in enumerate((b0, b1, b2, b3)):
            g = jnp.sum(oh2 * cb_ref[:, c, :], axis=1, keepdims=True)
            g = g * (float(W) if c % 2 == 0 else float(H))
            outs.append(jnp.where(sel, g, acc))
        v = jnp.where(iota2 == imin, -1.0, v)
        return (v, fs, fl, outs[0], outs[1], outs[2], outs[3])

    z = jnp.zeros((B, MAX_DET), jnp.float32)
    _, fs, fl, b0, b1, b2, b3 = jax.lax.fori_loop(
        0, MAX_DET, sel_step, (fs_all, z, z, z, z, z, z))
    fs_ref[...] = fs
    fl_ref[...] = fl.astype(jnp.int32)
    fb_ref[:, 0, :] = b0
    fb_ref[:, 1, :] = b1
    fb_ref[:, 2, :] = b2
    fb_ref[:, 3, :] = b3


def kernel(feat0, feat1, feat2, feat3, feat4,
           cls_w0, cls_w1, cls_w2, cls_w3, cls_w4,
           cls_b0, cls_b1, cls_b2, cls_b3, cls_b4,
           reg_w0, reg_w1, reg_w2, reg_w3, reg_w4,
           reg_b0, reg_b1, reg_b2, reg_b3, reg_b4):
    feats = [feat0, feat1, feat2, feat3, feat4]
    cws = [cls_w0, cls_w1, cls_w2, cls_w3, cls_w4]
    cbs = [cls_b0, cls_b1, cls_b2, cls_b3, cls_b4]
    rws = [reg_w0, reg_w1, reg_w2, reg_w3, reg_w4]
    rbs = [reg_b0, reg_b1, reg_b2, reg_b3, reg_b4]

    xs = [_im2col(f) for f in feats]
    ws = [jnp.transpose(jnp.concatenate([cw, rw], 0),
                        (0, 2, 3, 1)).reshape(-1, KDIM)
          for cw, rw in zip(cws, rws)]
    bs = [jnp.concatenate([cb, rb], 0)[:, None]
          for cb, rb in zip(cbs, rbs)]

    outs = pl.pallas_call(
        _conv_kernel,
        out_shape=[jax.ShapeDtypeStruct((w.shape[0], x.shape[1]), jnp.float32)
                   for w, x in zip(ws, xs)],
    )(*xs, *ws, *bs)

    cls_parts, reg_parts = [], []
    for o, bpl, f in zip(outs, BPL, FM):
        p = f * f
        o = o.reshape(o.shape[0], B, p)
        cl = o[:bpl * NC].reshape(bpl, NC, B, p)
        cl = jnp.transpose(cl, (2, 1, 3, 0)).reshape(B, NC, p * bpl)
        rg = o[bpl * NC:].reshape(bpl, 4, B, p)
        rg = jnp.transpose(rg, (2, 1, 3, 0)).reshape(B, 4, p * bpl)
        cls_parts.append(cl)
        reg_parts.append(rg)
    cls_t = jnp.concatenate(cls_parts, axis=2)
    reg_t = jnp.concatenate(reg_parts, axis=2)
    padn = NPRI_PAD - NPRI
    cls_t = jnp.pad(cls_t, ((0, 0), (0, 0), (0, padn)))
    reg_t = jnp.pad(reg_t, ((0, 0), (0, 0), (0, padn)))
    pri_a = jnp.asarray(_PRI.T)
    pri_t = jnp.pad(jnp.concatenate(
        [pri_a, CV * pri_a[2:3], CV * pri_a[3:4]], 0),
        ((0, 0), (0, padn)))

    sc, boxes = pl.pallas_call(
        _head_kernel,
        out_shape=[jax.ShapeDtypeStruct((B, NC - 1, NPRI_PAD), jnp.float32),
                   jax.ShapeDtypeStruct((B, 4, NPRI_PAD), jnp.float32)],
    )(cls_t, reg_t, pri_t)

    flat = jnp.transpose(sc[:, :, :NPRI], (0, 2, 1)).reshape(B, NFLAT)
    flat = jnp.pad(flat, ((0, 0), (0, NFLAT_PAD - NFLAT)), constant_values=-1.0)

    ts, ti = pl.pallas_call(
        _topk_kernel,
        out_shape=[jax.ShapeDtypeStruct((B, CAND_K), jnp.float32),
                   jax.ShapeDtypeStruct((B, CAND_K), jnp.int32)],
        scratch_shapes=[pltpu.VMEM((B, NCHUNK, 128), jnp.float32)],
    )(flat.reshape(B, NCHUNK, 128))

    fbt, fl, fs = pl.pallas_call(
        _nms_kernel,
        out_shape=[jax.ShapeDtypeStruct((B, 4, MAX_DET), jnp.float32),
                   jax.ShapeDtypeStruct((B, MAX_DET), jnp.int32),
                   jax.ShapeDtypeStruct((B, MAX_DET), jnp.float32)],
        scratch_shapes=[pltpu.VMEM((B, CAND_K, NPRI_PAD), jnp.float32),
                        pltpu.VMEM((B, CAND_K, CAND_K), jnp.float32),
                        pltpu.VMEM((B, 4, CAND_K), jnp.float32)],
    )(ts, ti, boxes)
    fb = jnp.transpose(fbt, (0, 2, 1))
    return fb, fl, fs


# speedup vs baseline: 2.9390x; 2.9390x over previous
import numpy as np
import jax
import jax.numpy as jnp
from jax.experimental import pallas as pl
from jax.experimental.pallas import tpu as pltpu

H = 512
W = 512
NC = 21
BPL = [6, 6, 6, 4, 4]
FM = [16, 8, 4, 2, 1]
CAND_K = 200
MAX_DET = 100
CONF_TH = 0.01
NMS_TH = 0.45
CV, SV = 0.1, 0.2
B = 8

NPRI = sum(f * f * b for f, b in zip(FM, BPL))
NPRI_PAD = 2048
NFLAT = NPRI * (NC - 1)
NFLAT_PAD = 40960
KDIM = 256 * 9


def _priors_np():
    strides_x = [32, 64, 128, 213, 320]
    strides_y = [32, 64, 128, 256, 512]
    expand = H / 256.0
    mins = [10 * expand, 62 * expand, 114 * expand, 166 * expand, 218 * expand]
    maxs = [62 * expand, 114 * expand, 166 * expand, 218 * expand, 270 * expand]
    ars = [[2, 3], [2, 3], [2, 3], [2], [2]]
    pri = []
    for k in range(5):
        fy, fx = int(H / strides_y[k]), int(W / strides_x[k])
        for i in range(fy):
            for j in range(fx):
                cx, cy = (j + 0.5) / fx, (i + 0.5) / fy
                w0, h0 = mins[k] / W, mins[k] / H
                pri.append([cx, cy, w0, h0])
                s2 = float(np.sqrt(mins[k] * maxs[k]))
                pri.append([cx, cy, s2 / W, s2 / H])
                for r in ars[k]:
                    rt = float(np.sqrt(r))
                    pri.append([cx, cy, w0 * rt, h0 / rt])
                    pri.append([cx, cy, w0 / rt, h0 * rt])
    return np.clip(np.array(pri, np.float32), 0.0, 1.0)


_PRI = _priors_np()
assert _PRI.shape[0] == NPRI


def _prep_x(feat):
    b, ch, f, _ = feat.shape
    fp = f + 2
    xp = jnp.pad(feat, ((0, 0), (0, 0), (1, 1), (1, 1)))
    xt = jnp.transpose(xp, (1, 0, 2, 3)).reshape(ch, b * fp * fp)
    return jnp.pad(xt, ((0, 0), (0, 2 * fp + 2 + 6)))


def _conv_kernel(*refs):
    xs = refs[0:5]
    ws = refs[5:10]
    bs = refs[10:15]
    os_ = refs[15:20]
    for x, w, bb, o, f in zip(xs, ws, bs, os_, FM):
        fp = f + 2
        cols = B * fp * fp
        acc = None
        for di in range(3):
            for dj in range(3):
                t = jax.lax.dot_general(
                    w[di, dj], x[:, pl.ds(di * fp + dj, cols)],
                    (((1,), (0,)), ((), ())),
                    preferred_element_type=jnp.float32)
                acc = t if acc is None else acc + t
        o[...] = acc + bb[...]


def _head_kernel(cls_ref, reg_ref, pri_ref, sc_ref, box_ref):
    c = cls_ref[...]
    m = c[:, 0:1, :]
    for i in range(1, NC):
        m = jnp.maximum(m, c[:, i:i + 1, :])
    e = jnp.exp(c - m)
    s = e[:, 0:1, :]
    for i in range(1, NC):
        s = s + e[:, i:i + 1, :]
    p = e / s
    sc = p[:, 1:, :]
    pio = jax.lax.broadcasted_iota(jnp.int32, (B, NC - 1, NPRI_PAD), 2)
    sc_ref[...] = jnp.where((sc > CONF_TH) & (pio < NPRI), sc, 0.0)

    r = reg_ref[...]
    pcx = pri_ref[0:1, :]
    pcy = pri_ref[1:2, :]
    pw = pri_ref[2:3, :]
    ph = pri_ref[3:4, :]
    cvpw = pri_ref[4:5, :]
    cvph = pri_ref[5:6, :]
    cx = r[:, 0, :] * cvpw + pcx
    cy = r[:, 1, :] * cvph + pcy
    wd = jnp.exp(r[:, 2, :] * SV) * pw
    hg = jnp.exp(r[:, 3, :] * SV) * ph
    box_ref[:, 0, :] = cx - wd / 2.0
    box_ref[:, 1, :] = cy - hg / 2.0
    box_ref[:, 2, :] = cx + wd / 2.0
    box_ref[:, 3, :] = cy + hg / 2.0


def _topk_kernel(sc_ref, ts_ref, ti_ref, v_ref):
    v_ref[...] = sc_ref[...]
    cio = jax.lax.broadcasted_iota(jnp.int32, (B, NC - 1, NPRI_PAD), 1)
    pio = jax.lax.broadcasted_iota(jnp.int32, (B, NC - 1, NPRI_PAD), 2)
    fidx = pio * (NC - 1) + cio
    kio = jax.lax.broadcasted_iota(jnp.int32, (B, CAND_K), 1)

    def step(j, carry):
        ts, ti = carry
        v = v_ref[...]
        m1 = jnp.max(v, axis=1, keepdims=True)
        m = jnp.max(m1, axis=2, keepdims=True)
        cand = jnp.where(v == m, fidx, NFLAT_PAD * 20)
        i1 = jnp.min(cand, axis=1, keepdims=True)
        imin = jnp.min(i1, axis=2, keepdims=True)
        sel = kio == j
        ts = jnp.where(sel, m[:, :, 0], ts)
        ti = jnp.where(sel, imin[:, :, 0], ti)
        v_ref[...] = jnp.where(fidx == imin, -1.0, v)
        return ts, ti

    ts, ti = jax.lax.fori_loop(
        0, CAND_K, step,
        (jnp.zeros((B, CAND_K), jnp.float32),
         jnp.zeros((B, CAND_K), jnp.int32)))
    ts_ref[...] = ts
    ti_ref[...] = ti


def _nms_kernel(ts_ref, ti_ref, box_ref, fb_ref, fl_ref, fs_ref,
                oh_ref, sup_ref, cb_ref):
    ti = ti_ref[...]
    ts = ts_ref[...]
    pidx = ti // (NC - 1)
    label_f = (ti % (NC - 1) + 1).astype(jnp.float32)

    lane = jax.lax.broadcasted_iota(jnp.int32, (B, CAND_K, NPRI_PAD), 2)
    oh_ref[...] = (lane == pidx[:, :, None]).astype(jnp.float32)
    oh = oh_ref[...]
    for c in range(4):
        bc = box_ref[:, c, :][:, None, :]
        cb_ref[:, c, :] = jnp.sum(oh * bc, axis=2)

    nb = [cb_ref[:, c, :] + label_f * 2.0 for c in range(4)]
    ltx = jnp.maximum(nb[0][:, :, None], nb[0][:, None, :])
    lty = jnp.maximum(nb[1][:, :, None], nb[1][:, None, :])
    rbx = jnp.minimum(nb[2][:, :, None], nb[2][:, None, :])
    rby = jnp.minimum(nb[3][:, :, None], nb[3][:, None, :])
    w_i = jnp.clip(rbx - ltx, 0.0)
    h_i = jnp.clip(rby - lty, 0.0)
    inter = w_i * h_i
    area = (nb[2] - nb[0]) * (nb[3] - nb[1])
    iou = inter / (area[:, :, None] + area[:, None, :] - inter + 1e-9)
    jgti = (jax.lax.broadcasted_iota(jnp.int32, (B, CAND_K, CAND_K), 2) >
            jax.lax.broadcasted_iota(jnp.int32, (B, CAND_K, CAND_K), 1))
    sup_ref[...] = jnp.where((iou > NMS_TH) & jgti, 1.0, 0.0)

    iota2 = jax.lax.broadcasted_iota(jnp.int32, (B, CAND_K), 1)

    def nms_step(i, keep):
        row = sup_ref[:, pl.ds(i, 1), :].reshape(B, CAND_K)
        ki = jnp.sum(jnp.where(iota2 == i, keep, 0.0), axis=1, keepdims=True)
        return keep * (1.0 - row * ki)

    keep = jax.lax.fori_loop(0, CAND_K, nms_step,
                             jnp.ones((B, CAND_K), jnp.float32))

    fs_all = ts * keep
    dio = jax.lax.broadcasted_iota(jnp.int32, (B, MAX_DET), 1)

    def sel_step(j, carry):
        v, fs, fl, b0, b1, b2, b3 = carry
        m = jnp.max(v, axis=1, keepdims=True)
        idx = jnp.where(v == m, iota2, CAND_K)
        imin = jnp.min(idx, axis=1, keepdims=True)
        oh2 = (iota2 == imin).astype(jnp.float32)
        sel = dio == j
        fs = jnp.where(sel, m, fs)
        lab = jnp.sum(oh2 * label_f, axis=1, keepdims=True)
        fl = jnp.where(sel, lab, fl)
        outs = []
        for c, acc in enumerate((b0, b1, b2, b3)):
            g = jnp.sum(oh2 * cb_ref[:, c, :], axis=1, keepdims=True)
            g = g * (float(W) if c % 2 == 0 else float(H))
            outs.append(jnp.where(sel, g, acc))
        v = jnp.where(iota2 == imin, -1.0, v)
        return (v, fs, fl, outs[0], outs[1], outs[2], outs[3])

    z = jnp.zeros((B, MAX_DET), jnp.float32)
    _, fs, fl, b0, b1, b2, b3 = jax.lax.fori_loop(
        0, MAX_DET, sel_step, (fs_all, z, z, z, z, z, z))
    fs_ref[...] = fs
    fl_ref[...] = fl.astype(jnp.int32)
    fb_ref[:, 0, :] = b0
    fb_ref[:, 1, :] = b1
    fb_ref[:, 2, :] = b2
    fb_ref[:, 3, :] = b3


def kernel(feat0, feat1, feat2, feat3, feat4,
           cls_w0, cls_w1, cls_w2, cls_w3, cls_w4,
           cls_b0, cls_b1, cls_b2, cls_b3, cls_b4,
           reg_w0, reg_w1, reg_w2, reg_w3, reg_w4,
           reg_b0, reg_b1, reg_b2, reg_b3, reg_b4):
    feats = [feat0, feat1, feat2, feat3, feat4]
    cws = [cls_w0, cls_w1, cls_w2, cls_w3, cls_w4]
    cbs = [cls_b0, cls_b1, cls_b2, cls_b3, cls_b4]
    rws = [reg_w0, reg_w1, reg_w2, reg_w3, reg_w4]
    rbs = [reg_b0, reg_b1, reg_b2, reg_b3, reg_b4]

    xs = [_prep_x(f) for f in feats]
    ws = [jnp.transpose(jnp.concatenate([cw, rw], 0), (2, 3, 0, 1))
          for cw, rw in zip(cws, rws)]
    bs = [jnp.concatenate([cb, rb], 0)[:, None]
          for cb, rb in zip(cbs, rbs)]

    outs = pl.pallas_call(
        _conv_kernel,
        out_shape=[jax.ShapeDtypeStruct(
            (w.shape[2], B * (f + 2) * (f + 2)), jnp.float32)
            for w, f in zip(ws, FM)],
    )(*xs, *ws, *bs)

    cls_parts, reg_parts = [], []
    for o, bpl, f in zip(outs, BPL, FM):
        p = f * f
        fp = f + 2
        o = o.reshape(o.shape[0], B, fp, fp)[:, :, :f, :f]
        o = o.reshape(o.shape[0], B, p)
        cl = o[:bpl * NC].reshape(bpl, NC, B, p)
        cl = jnp.transpose(cl, (2, 1, 3, 0)).reshape(B, NC, p * bpl)
        rg = o[bpl * NC:].reshape(bpl, 4, B, p)
        rg = jnp.transpose(rg, (2, 1, 3, 0)).reshape(B, 4, p * bpl)
        cls_parts.append(cl)
        reg_parts.append(rg)
    cls_t = jnp.concatenate(cls_parts, axis=2)
    reg_t = jnp.concatenate(reg_parts, axis=2)
    padn = NPRI_PAD - NPRI
    cls_t = jnp.pad(cls_t, ((0, 0), (0, 0), (0, padn)))
    reg_t = jnp.pad(reg_t, ((0, 0), (0, 0), (0, padn)))
    pri_a = jnp.asarray(_PRI.T)
    pri_t = jnp.pad(jnp.concatenate(
        [pri_a, CV * pri_a[2:3], CV * pri_a[3:4]], 0),
        ((0, 0), (0, padn)))

    sc, boxes = pl.pallas_call(
        _head_kernel,
        out_shape=[jax.ShapeDtypeStruct((B, NC - 1, NPRI_PAD), jnp.float32),
                   jax.ShapeDtypeStruct((B, 4, NPRI_PAD), jnp.float32)],
    )(cls_t, reg_t, pri_t)

    ts, ti = pl.pallas_call(
        _topk_kernel,
        out_shape=[jax.ShapeDtypeStruct((B, CAND_K), jnp.float32),
                   jax.ShapeDtypeStruct((B, CAND_K), jnp.int32)],
        scratch_shapes=[pltpu.VMEM((B, NC - 1, NPRI_PAD), jnp.float32)],
    )(sc)

    fbt, fl, fs = pl.pallas_call(
        _nms_kernel,
        out_shape=[jax.ShapeDtypeStruct((B, 4, MAX_DET), jnp.float32),
                   jax.ShapeDtypeStruct((B, MAX_DET), jnp.int32),
                   jax.ShapeDtypeStruct((B, MAX_DET), jnp.float32)],
        scratch_shapes=[pltpu.VMEM((B, CAND_K, NPRI_PAD), jnp.float32),
                        pltpu.VMEM((B, CAND_K, CAND_K), jnp.float32),
                        pltpu.VMEM((B, 4, CAND_K), jnp.float32)],
    )(ts, ti, boxes)
    fb = jnp.transpose(fbt, (0, 2, 1))
    return fb, fl, fs
